# baseline (speedup 1.0000x reference)
import functools

import jax
import jax.numpy as jnp
from jax import lax
from jax.experimental import pallas as pl
from jax.experimental.pallas import tpu as pltpu
from jax.experimental.pallas import tpu_sc as plsc

_NUM_CORES = 2
_NUM_SUBCORES = 16
_NUM_WORKERS = _NUM_CORES * _NUM_SUBCORES
_LANES = 16
_CHUNK = 256


def _make_gather(batch, n_fields, depth):
    per_w = batch // _NUM_WORKERS
    n_sub = per_w // _CHUNK
    n_chunks = n_fields * n_sub
    assert per_w % _CHUNK == 0 and n_chunks % 2 == 0

    mesh = plsc.VectorSubcoreMesh(
        core_axis_name="c",
        subcore_axis_name="s",
        num_cores=_NUM_CORES,
        num_subcores=_NUM_SUBCORES,
    )

    @functools.partial(
        pl.kernel,
        out_type=jax.ShapeDtypeStruct((n_fields, depth, batch), jnp.float32),
        mesh=mesh,
        scratch_types=[
            pltpu.VMEM((_CHUNK,), jnp.int32),
            pltpu.VMEM((_CHUNK,), jnp.int32),
            pltpu.VMEM((_CHUNK, 2 * depth), jnp.float32),
            pltpu.VMEM((_CHUNK, 2 * depth), jnp.float32),
            pltpu.VMEM((depth, _CHUNK), jnp.float32),
            pltpu.VMEM((depth, _CHUNK), jnp.float32),
            pltpu.SemaphoreType.DMA,
            pltpu.SemaphoreType.DMA,
            pltpu.SemaphoreType.DMA,
            pltpu.SemaphoreType.DMA,
        ],
        compiler_params=pltpu.CompilerParams(
            use_tc_tiling_on_sc=True, needs_layout_passes=False),
    )
    def gather_kernel(idxt_hbm, table_hbm, out_hbm,
                      idx0, idx1, rows0, rows1, tb0, tb1,
                      gsem0, gsem1, wsem0, wsem1):
        wid = lax.axis_index("s") * _NUM_CORES + lax.axis_index("c")
        b_base = wid * per_w
        idxs = (idx0, idx1)
        rows = (rows0, rows1)
        tbs = (tb0, tb1)
        gsems = (gsem0, gsem1)
        wsems = (wsem0, wsem1)

        def chunk_b0(i):
            return b_base + (i % n_sub) * _CHUNK

        def start_gather(i, b):
            f = i // n_sub
            pltpu.sync_copy(idxt_hbm.at[f, pl.ds(chunk_b0(i), _CHUNK)],
                            idxs[b])
            pltpu.async_copy(table_hbm.at[idxs[b]], rows[b], gsems[b])

        def wait_gather(b):
            pltpu.make_async_copy(table_hbm.at[idxs[b]], rows[b],
                                  gsems[b]).wait()

        def start_write(i, b):
            f = i // n_sub
            pltpu.async_copy(tbs[b],
                             out_hbm.at[f, pl.ds(0, depth),
                                        pl.ds(chunk_b0(i), _CHUNK)],
                             wsems[b])

        def wait_write(b):
            pltpu.make_async_copy(tbs[b],
                                  out_hbm.at[0, pl.ds(0, depth),
                                             pl.ds(b_base, _CHUNK)],
                                  wsems[b]).wait()

        def transpose(b):
            for j in range(_CHUNK // _LANES):
                rowvec = lax.iota(jnp.int32, _LANES) + j * _LANES
                for d in range(depth):
                    colvec = jnp.full((_LANES,), d, jnp.int32)
                    vals = plsc.load_gather(rows[b], [rowvec, colvec])
                    tbs[b][d, pl.ds(j * _LANES, _LANES)] = vals

        start_gather(0, 0)

        def body(j, carry):
            for b in range(2):
                i = 2 * j + b
                nb = 1 - b

                @pl.when(i >= 1)
                def _():
                    wait_write(nb)

                @pl.when(i + 1 < n_chunks)
                def _():
                    start_gather(i + 1, nb)

                wait_gather(b)
                transpose(b)
                start_write(i, b)
            return carry

        lax.fori_loop(0, n_chunks // 2, body, 0)
        wait_write(1)

    return gather_kernel


def kernel(indices, embedding):
    batch, n_fields = indices.shape
    n_rows, depth = embedding.shape
    idxt = indices.T.astype(jnp.int32)
    table128 = jnp.concatenate(
        [embedding, jnp.zeros((n_rows, depth), embedding.dtype)], axis=1)
    out = _make_gather(batch, n_fields, depth)(idxt, table128)
    return jnp.transpose(out, (2, 0, 1))


# speedup vs baseline: 1.1847x; 1.1847x over previous
import functools

import jax
import jax.numpy as jnp
from jax import lax
from jax.experimental import pallas as pl
from jax.experimental.pallas import tpu as pltpu
from jax.experimental.pallas import tpu_sc as plsc

_NUM_CORES = 2
_NUM_SUBCORES = 16
_NUM_WORKERS = _NUM_CORES * _NUM_SUBCORES
_LANES = 16
_CHUNK = 256


def _make_gather(batch, n_fields, depth):
    per_w = batch // _NUM_WORKERS
    n_sub = per_w // _CHUNK
    n_chunks = n_fields * n_sub
    assert per_w % _CHUNK == 0 and n_chunks % 2 == 0

    mesh = plsc.VectorSubcoreMesh(
        core_axis_name="c",
        subcore_axis_name="s",
        num_cores=_NUM_CORES,
        num_subcores=_NUM_SUBCORES,
    )

    @functools.partial(
        pl.kernel,
        out_type=jax.ShapeDtypeStruct((n_fields, depth, batch), jnp.float32),
        mesh=mesh,
        scratch_types=[
            pltpu.VMEM((_CHUNK,), jnp.int32),
            pltpu.VMEM((_CHUNK,), jnp.int32),
            pltpu.VMEM((_CHUNK, 2 * depth), jnp.float32),
            pltpu.VMEM((_CHUNK, 2 * depth), jnp.float32),
            pltpu.VMEM((depth, _CHUNK), jnp.float32),
            pltpu.VMEM((depth, _CHUNK), jnp.float32),
            pltpu.SemaphoreType.DMA,
            pltpu.SemaphoreType.DMA,
            pltpu.SemaphoreType.DMA,
            pltpu.SemaphoreType.DMA,
        ],
        compiler_params=pltpu.CompilerParams(
            use_tc_tiling_on_sc=True, needs_layout_passes=False),
    )
    def gather_kernel(idxt_hbm, table_hbm, out_hbm,
                      idx0, idx1, rows0, rows1, tb0, tb1,
                      gsem0, gsem1, wsem0, wsem1):
        wid = lax.axis_index("s") * _NUM_CORES + lax.axis_index("c")
        b_base = wid * per_w
        idxs = (idx0, idx1)
        rows = (rows0, rows1)
        tbs = (tb0, tb1)
        gsems = (gsem0, gsem1)
        wsems = (wsem0, wsem1)

        def chunk_b0(i):
            return b_base + (i % n_sub) * _CHUNK

        def start_gather(i, b):
            f = i // n_sub
            pltpu.sync_copy(idxt_hbm.at[f, pl.ds(chunk_b0(i), _CHUNK)],
                            idxs[b])
            pltpu.async_copy(table_hbm.at[idxs[b]], rows[b], gsems[b])

        def wait_gather(b):
            pltpu.make_async_copy(table_hbm.at[idxs[b]], rows[b],
                                  gsems[b]).wait()

        def start_write(i, b):
            f = i // n_sub
            pltpu.async_copy(tbs[b],
                             out_hbm.at[f, pl.ds(0, depth),
                                        pl.ds(chunk_b0(i), _CHUNK)],
                             wsems[b])

        def wait_write(b):
            pltpu.make_async_copy(tbs[b],
                                  out_hbm.at[0, pl.ds(0, depth),
                                             pl.ds(b_base, _CHUNK)],
                                  wsems[b]).wait()

        n_j = _CHUNK // _LANES
        iota = lax.iota(jnp.int32, _LANES)

        def transpose(b):
            @plsc.parallel_loop(0, depth * n_j, 1, unroll=4)
            def _(q):
                d = q // n_j
                j = q % n_j
                rowvec = iota + j * _LANES
                colvec = jnp.broadcast_to(d, (_LANES,))
                vals = plsc.load_gather(rows[b], [rowvec, colvec])
                tbs[b][d, pl.ds(j * _LANES, _LANES)] = vals

        start_gather(0, 0)

        def body(j, carry):
            for b in range(2):
                i = 2 * j + b
                nb = 1 - b

                @pl.when(i >= 1)
                def _():
                    wait_write(nb)

                @pl.when(i + 1 < n_chunks)
                def _():
                    start_gather(i + 1, nb)

                wait_gather(b)
                transpose(b)
                start_write(i, b)
            return carry

        lax.fori_loop(0, n_chunks // 2, body, 0)
        wait_write(1)

    return gather_kernel


def kernel(indices, embedding):
    batch, n_fields = indices.shape
    n_rows, depth = embedding.shape
    idxt = indices.T.astype(jnp.int32)
    table128 = jnp.concatenate(
        [embedding, jnp.zeros((n_rows, depth), embedding.dtype)], axis=1)
    out = _make_gather(batch, n_fields, depth)(idxt, table128)
    return jnp.transpose(out, (2, 0, 1))


# speedup vs baseline: 1.3859x; 1.1699x over previous
import functools

import jax
import jax.numpy as jnp
from jax import lax
from jax.experimental import pallas as pl
from jax.experimental.pallas import tpu as pltpu
from jax.experimental.pallas import tpu_sc as plsc

_NUM_CORES = 2
_NUM_SUBCORES = 16
_NUM_WORKERS = _NUM_CORES * _NUM_SUBCORES
_LANES = 16
_CHUNK = 256


def _make_gather(batch, n_fields, depth):
    per_w = batch // _NUM_WORKERS
    n_sub = per_w // _CHUNK
    n_chunks = n_fields * n_sub
    assert per_w % _CHUNK == 0 and n_chunks % 2 == 0

    mesh = plsc.VectorSubcoreMesh(
        core_axis_name="c",
        subcore_axis_name="s",
        num_cores=_NUM_CORES,
        num_subcores=_NUM_SUBCORES,
    )

    @functools.partial(
        pl.kernel,
        out_type=jax.ShapeDtypeStruct((n_fields, depth, batch), jnp.float32),
        mesh=mesh,
        scratch_types=[
            pltpu.VMEM((_CHUNK,), jnp.int32),
            pltpu.VMEM((_CHUNK,), jnp.int32),
            pltpu.VMEM((_CHUNK, 2 * depth), jnp.float32),
            pltpu.VMEM((_CHUNK, 2 * depth), jnp.float32),
            pltpu.VMEM((depth, _CHUNK + 1), jnp.float32),
            pltpu.VMEM((depth, _CHUNK + 1), jnp.float32),
            pltpu.SemaphoreType.DMA,
            pltpu.SemaphoreType.DMA,
            pltpu.SemaphoreType.DMA,
            pltpu.SemaphoreType.DMA,
        ],
        compiler_params=pltpu.CompilerParams(
            use_tc_tiling_on_sc=True, needs_layout_passes=False),
    )
    def gather_kernel(idxt_hbm, table_hbm, out_hbm,
                      idx0, idx1, rows0, rows1, tb0, tb1,
                      gsem0, gsem1, wsem0, wsem1):
        wid = lax.axis_index("s") * _NUM_CORES + lax.axis_index("c")
        b_base = wid * per_w
        idxs = (idx0, idx1)
        rows = (rows0, rows1)
        tbs = (tb0, tb1)
        gsems = (gsem0, gsem1)
        wsems = (wsem0, wsem1)

        def chunk_b0(i):
            return b_base + (i % n_sub) * _CHUNK

        def start_gather(i, b):
            f = i // n_sub
            pltpu.sync_copy(idxt_hbm.at[f, pl.ds(chunk_b0(i), _CHUNK)],
                            idxs[b])
            pltpu.async_copy(table_hbm.at[idxs[b]], rows[b], gsems[b])

        def wait_gather(b):
            pltpu.make_async_copy(table_hbm.at[idxs[b]], rows[b],
                                  gsems[b]).wait()

        def start_write(i, b):
            f = i // n_sub
            pltpu.async_copy(tbs[b].at[:, pl.ds(0, _CHUNK)],
                             out_hbm.at[f, pl.ds(0, depth),
                                        pl.ds(chunk_b0(i), _CHUNK)],
                             wsems[b])

        def wait_write(b):
            pltpu.make_async_copy(tbs[b].at[:, pl.ds(0, _CHUNK)],
                                  out_hbm.at[0, pl.ds(0, depth),
                                             pl.ds(b_base, _CHUNK)],
                                  wsems[b]).wait()

        iota = lax.iota(jnp.int32, _LANES)

        def transpose(b):
            @plsc.parallel_loop(0, _CHUNK, 1, unroll=4)
            def _(r):
                bvec = jnp.broadcast_to(r, (_LANES,))
                for k in range(depth // _LANES):
                    vals = rows[b][r, pl.ds(k * _LANES, _LANES)]
                    plsc.store_scatter(tbs[b], [iota + k * _LANES, bvec],
                                       vals)

        start_gather(0, 0)

        def body(j, carry):
            for b in range(2):
                i = 2 * j + b
                nb = 1 - b

                @pl.when(i >= 1)
                def _():
                    wait_write(nb)

                @pl.when(i + 1 < n_chunks)
                def _():
                    start_gather(i + 1, nb)

                wait_gather(b)
                transpose(b)
                start_write(i, b)
            return carry

        lax.fori_loop(0, n_chunks // 2, body, 0)
        wait_write(1)

    return gather_kernel


def kernel(indices, embedding):
    batch, n_fields = indices.shape
    n_rows, depth = embedding.shape
    idxt = indices.T.astype(jnp.int32)
    table128 = jnp.concatenate(
        [embedding, jnp.zeros((n_rows, depth), embedding.dtype)], axis=1)
    out = _make_gather(batch, n_fields, depth)(idxt, table128)
    return jnp.transpose(out, (2, 0, 1))
